# Initial kernel scaffold; baseline (speedup 1.0000x reference)
import jax
import jax.numpy as jnp
from jax.experimental import pallas as pl


def kernel(x, e, ratio, Wqu, bqu, Wqv, bqv):
    raise NotImplementedError("write your pallas kernel here")


# speedup vs baseline: 2.2198x; 2.2198x over previous
import functools

import jax
import jax.numpy as jnp
from jax import lax
from jax.experimental import pallas as pl
from jax.experimental.pallas import tpu as pltpu
from jax.experimental.pallas import tpu_sc as plsc

N_CORES = 2
N_SUBCORES = 16
N_WORKERS = N_CORES * N_SUBCORES
LANES = 16
BLK = 128


def _proj_body(x_ref, wu_ref, bu_ref, wv_ref, bv_ref, ku_ref, kv_ref):
    x = x_ref[...]
    ku_ref[...] = jnp.dot(x, wu_ref[...], preferred_element_type=jnp.float32) + bu_ref[...]
    kv_ref[...] = jnp.dot(x, wv_ref[...], preferred_element_type=jnp.float32) + bv_ref[...]


def _edge_body(ku_hbm, kv_hbm, e0_hbm, e1_hbm, hs_hbm, out_hbm,
               idx0_v, idx1_v, a_v, b_v, out_v, hs_v, sem):
    nblocks = e0_hbm.shape[0]
    wid = lax.axis_index("s") * N_CORES + lax.axis_index("c")
    nsteps = (nblocks + N_WORKERS - 1) // N_WORKERS

    pltpu.sync_copy(hs_hbm, hs_v)
    hsv = hs_v[...]
    lane = lax.iota(jnp.int32, LANES)
    sh1 = lane ^ 1
    sh2 = lane ^ 2
    place = [((lane - 4 * k) << 2) & 15 for k in range(4)]
    msk = [(lane >> 2) == k for k in range(4)]
    one = jnp.full((LANES,), 1.0, jnp.float32)

    def step(t, _):
        b = wid + t * N_WORKERS

        @pl.when(b < nblocks)
        def _():
            pltpu.sync_copy(e0_hbm.at[b], idx0_v)
            pltpu.sync_copy(e1_hbm.at[b], idx1_v)
            c1 = pltpu.async_copy(ku_hbm.at[idx0_v], a_v, sem)
            c2 = pltpu.async_copy(kv_hbm.at[idx1_v], b_v, sem)
            c1.wait()
            c2.wait()

            def edge4(i, _):
                acc = one
                for k in range(4):
                    p = a_v[4 * i + k] * b_v[4 * i + k]
                    t1 = p + p.at[sh1].get(mode="promise_in_bounds")
                    s = t1 + t1.at[sh2].get(mode="promise_in_bounds")
                    d = s.at[place[k]].get(mode="promise_in_bounds")
                    acc = jnp.where(msk[k], d, acc)
                out_v[i] = hsv / (one + jnp.exp(-acc)) + one
                return 0

            lax.fori_loop(0, BLK // 4, edge4, 0)
            pltpu.sync_copy(out_v, out_hbm.at[pl.ds(b * (BLK // 4), BLK // 4), :])

        return 0

    lax.fori_loop(0, nsteps, step, 0)


def kernel(x, e, ratio, Wqu, bqu, Wqv, bqv):
    n, h = x.shape
    out_dim = Wqu.shape[1]
    num_e = e.shape[1]
    stru = 4
    assert out_dim == LANES and num_e % BLK == 0

    ku, kv = pl.pallas_call(
        _proj_body,
        out_shape=(
            jax.ShapeDtypeStruct((n, out_dim), jnp.float32),
            jax.ShapeDtypeStruct((n, out_dim), jnp.float32),
        ),
    )(x, Wqu, bqu.reshape(1, out_dim), Wqv, bqv.reshape(1, out_dim))

    e0 = e[0].reshape(num_e // BLK, BLK).astype(jnp.int32)
    e1 = e[1].reshape(num_e // BLK, BLK).astype(jnp.int32)
    hs = jnp.full((LANES,), (1.0 - ratio) * 0.5, jnp.float32)

    mesh = plsc.VectorSubcoreMesh(core_axis_name="c", subcore_axis_name="s")
    edge_kernel = pl.kernel(
        _edge_body,
        out_type=jax.ShapeDtypeStruct((num_e // 4, LANES), jnp.float32),
        mesh=mesh,
        scratch_types=[
            pltpu.VMEM((BLK,), jnp.int32),
            pltpu.VMEM((BLK,), jnp.int32),
            pltpu.VMEM((BLK, LANES), jnp.float32),
            pltpu.VMEM((BLK, LANES), jnp.float32),
            pltpu.VMEM((BLK // 4, LANES), jnp.float32),
            pltpu.VMEM((LANES,), jnp.float32),
            pltpu.SemaphoreType.DMA,
        ],
        compiler_params=pltpu.CompilerParams(use_tc_tiling_on_sc=False),
    )
    out = edge_kernel(ku, kv, e0, e1, hs).reshape(num_e, stru)
    return tuple(out[:, i] for i in range(stru))
